# baseline (speedup 1.0000x reference)
import functools

import jax
import jax.numpy as jnp
from jax import lax
from jax.experimental import pallas as pl
from jax.experimental.pallas import tpu as pltpu
from jax.experimental.pallas import tpu_sc as plsc

N = 10000
E = 320000
NC = 2
NS = 16
EPW = E // (NC * NS)
CH = 125
NCH = EPW // CH
RPW = N // NS

_MESH = plsc.VectorSubcoreMesh(core_axis_name="c", subcore_axis_name="s")


@functools.partial(
    pl.kernel,
    mesh=_MESH,
    out_type=jax.ShapeDtypeStruct((NC, NS, RPW, 16), jnp.float32),
    scratch_types=[
        pltpu.VMEM((NCH, CH), jnp.int32),
        pltpu.VMEM((CH, 16), jnp.float32),
        pltpu.VMEM_SHARED((N, 16), jnp.float32),
        pltpu.SemaphoreType.DMA,
    ],
    compiler_params=pltpu.CompilerParams(use_tc_tiling_on_sc=False),
)
def _deg_pass(dst_hbm, ones_hbm, zero_hbm, out_hbm, idx_v, ones_v, acc_sh,
              dsem):
    c = lax.axis_index("c")
    s = lax.axis_index("s")
    pltpu.sync_copy(dst_hbm.at[c, s], idx_v)
    pltpu.sync_copy(ones_hbm, ones_v)
    pltpu.sync_copy(zero_hbm.at[s], acc_sh.at[pl.ds(s * RPW, RPW)])
    plsc.subcore_barrier()

    @pl.loop(0, NCH, step=8)
    def _(j):
        copies = [pltpu.async_copy(ones_v, acc_sh.at[idx_v.at[j + k]], dsem,
                                   add=True) for k in range(8)]
        for cp in copies:
            cp.wait()

    plsc.subcore_barrier()
    pltpu.sync_copy(acc_sh.at[pl.ds(s * RPW, RPW)], out_hbm.at[c, s])


@functools.partial(
    pl.kernel,
    mesh=_MESH,
    out_type=jax.ShapeDtypeStruct((NC, NS, RPW, 32), jnp.float32),
    scratch_types=[
        pltpu.VMEM((NCH, CH), jnp.int32),
        pltpu.VMEM((NCH, CH), jnp.int32),
        pltpu.VMEM((8 * CH, 32), jnp.float32),
        pltpu.VMEM((8 * CH, 32), jnp.float32),
        pltpu.VMEM_SHARED((N, 32), jnp.float32),
        pltpu.VMEM_SHARED((N, 32), jnp.float32),
        pltpu.SemaphoreType.DMA,
        pltpu.SemaphoreType.DMA,
        pltpu.SemaphoreType.DMA,
        pltpu.SemaphoreType.DMA,
    ],
    compiler_params=pltpu.CompilerParams(use_tc_tiling_on_sc=False),
)
def _spmm_pass(hs_hbm, src_hbm, dst_hbm, zero_hbm, out_hbm,
               src_v, dst_v, buf_a, buf_b, hs_sh, acc_sh, gsem_a, gsem_b,
               ssem_a, ssem_b):
    c = lax.axis_index("c")
    s = lax.axis_index("s")
    pltpu.sync_copy(src_hbm.at[c, s], src_v)
    pltpu.sync_copy(dst_hbm.at[c, s], dst_v)
    rows = pl.ds(s * RPW, RPW)
    pltpu.sync_copy(hs_hbm.at[rows], hs_sh.at[rows])
    pltpu.sync_copy(zero_hbm.at[s], acc_sh.at[rows])
    plsc.subcore_barrier()

    def _fire_g(b, buf, sem):
        for k in range(8):
            pltpu.async_copy(hs_sh.at[src_v.at[b * 8 + k]],
                             buf.at[pl.ds(k * CH, CH)], sem)

    def _fire_s(b, buf, sem):
        for k in range(8):
            pltpu.async_copy(buf.at[pl.ds(k * CH, CH)],
                             acc_sh.at[dst_v.at[b * 8 + k]], sem, add=True)

    def _drain(buf, sem):
        pltpu.make_async_copy(hs_hbm.at[pl.ds(0, 8 * CH)], buf, sem).wait()

    NB = NCH // 8
    _fire_g(0, buf_a, gsem_a)

    @pl.loop(0, NB, step=2)
    def _(b):
        _drain(buf_a, gsem_a)
        @pl.when(b + 1 < NB)
        def _():
            pl.when(b > 0)(lambda: _drain(buf_b, ssem_b))
            _fire_g(b + 1, buf_b, gsem_b)
        _fire_s(b, buf_a, ssem_a)
        @pl.when(b + 1 < NB)
        def _():
            _drain(buf_b, gsem_b)
            _drain(buf_a, ssem_a)
            pl.when(b + 2 < NB)(lambda: _fire_g(b + 2, buf_a, gsem_a))
            _fire_s(b + 1, buf_b, ssem_b)
    _drain(buf_b, ssem_b)

    plsc.subcore_barrier()
    pltpu.sync_copy(acc_sh.at[rows], out_hbm.at[c, s])


def _dinv_scale_body(deg_ref, x_ref, w_ref, dinv_ref, hs_ref):
    d = deg_ref[...]
    deg = d[0, :, 0:1] + d[1, :, 0:1] + 1.0
    dinv = lax.rsqrt(deg)
    dinv_ref[...] = dinv
    h = jnp.dot(x_ref[...], w_ref[...], preferred_element_type=jnp.float32)
    hs_ref[...] = h * dinv


def _layer_body(acc_ref, hs_ref, dinv_ref, b_ref, g_ref, be_ref, w_ref,
                o_ref):
    a = acc_ref[...]
    dinv = dinv_ref[...]
    t = (a[0] + a[1] + hs_ref[...]) * dinv + b_ref[...]
    t = jnp.maximum(t, 0.0)
    mu = jnp.mean(t, axis=1, keepdims=True)
    var = jnp.mean((t - mu) ** 2, axis=1, keepdims=True)
    t = (t - mu) * lax.rsqrt(var + 1e-5) * g_ref[...] + be_ref[...]
    o_ref[...] = jnp.dot(t, w_ref[...],
                         preferred_element_type=jnp.float32) * dinv


def _head_body(acc_ref, hs_ref, dinv_ref, b3_ref, wp1_ref, bp1_ref,
               wp2_ref, bp2_ref, emb_ref, o_ref):
    a = acc_ref[...]
    emb = (a[0] + a[1] + hs_ref[...]) * dinv_ref[...] + b3_ref[...]
    emb_ref[...] = emb
    h = jnp.maximum(emb, 0.0)
    h = jnp.dot(h, wp1_ref[...], preferred_element_type=jnp.float32)
    h = h + bp1_ref[...]
    q = jnp.dot(h, wp2_ref[...], preferred_element_type=jnp.float32)
    q = q + bp2_ref[...]
    m = jnp.max(q, axis=1, keepdims=True)
    shifted = q - m
    lse = jnp.log(jnp.sum(jnp.exp(shifted), axis=1, keepdims=True))
    o_ref[...] = shifted - lse


def _f32(*shape):
    return jax.ShapeDtypeStruct(shape, jnp.float32)


def kernel(x, edge_index, W1, b1, g1, be1, W2, b2, g2, be2, W3, b3,
           Wp1, bp1, Wp2, bp2):
    src_r = edge_index[0].reshape(NC, NS, NCH, CH)
    dst_r = edge_index[1].reshape(NC, NS, NCH, CH)
    ones16 = jnp.ones((CH, 16), jnp.float32)
    z16 = jnp.zeros((NS, RPW, 16), jnp.float32)
    z32 = jnp.zeros((NS, RPW, 32), jnp.float32)

    deg_acc = _deg_pass(dst_r, ones16, z16).reshape(NC, N, 16)
    dinv, hs1 = pl.pallas_call(
        _dinv_scale_body, out_shape=(_f32(N, 1), _f32(N, 32)))(deg_acc, x, W1)

    acc1 = _spmm_pass(hs1, src_r, dst_r, z32)
    hs2 = pl.pallas_call(_layer_body, out_shape=_f32(N, 32))(
        acc1.reshape(NC, N, 32), hs1, dinv, b1.reshape(1, 32),
        g1.reshape(1, 32), be1.reshape(1, 32), W2)

    acc2 = _spmm_pass(hs2, src_r, dst_r, z32)
    hs3 = pl.pallas_call(_layer_body, out_shape=_f32(N, 32))(
        acc2.reshape(NC, N, 32), hs2, dinv, b2.reshape(1, 32),
        g2.reshape(1, 32), be2.reshape(1, 32), W3)

    acc3 = _spmm_pass(hs3, src_r, dst_r, z32)
    emb, logp = pl.pallas_call(
        _head_body, out_shape=(_f32(N, 32), _f32(N, 128)))(
        acc3.reshape(NC, N, 32), hs3, dinv, b3.reshape(1, 32), Wp1,
        bp1.reshape(1, 32), Wp2, bp2.reshape(1, 128))
    return (emb, logp)


# speedup vs baseline: 1.0201x; 1.0201x over previous
import functools

import jax
import jax.numpy as jnp
from jax import lax
from jax.experimental import pallas as pl
from jax.experimental.pallas import tpu as pltpu
from jax.experimental.pallas import tpu_sc as plsc

N = 10000
E = 320000
NC = 2
NS = 16
EPW = E // (NC * NS)
CH = 125
NCH = EPW // CH
RPW = N // NS

_MESH = plsc.VectorSubcoreMesh(core_axis_name="c", subcore_axis_name="s")


@functools.partial(
    pl.kernel,
    mesh=_MESH,
    out_type=jax.ShapeDtypeStruct((NC, NS, RPW, 16), jnp.float32),
    scratch_types=[
        pltpu.VMEM((NCH, CH), jnp.int32),
        pltpu.VMEM((CH, 16), jnp.float32),
        pltpu.VMEM_SHARED((N, 16), jnp.float32),
        pltpu.SemaphoreType.DMA,
    ],
    compiler_params=pltpu.CompilerParams(use_tc_tiling_on_sc=False, skip_device_barrier=True),
)
def _deg_pass(dst_hbm, ones_hbm, zero_hbm, out_hbm, idx_v, ones_v, acc_sh,
              dsem):
    c = lax.axis_index("c")
    s = lax.axis_index("s")
    pltpu.sync_copy(dst_hbm.at[c, s], idx_v)
    pltpu.sync_copy(ones_hbm, ones_v)
    pltpu.sync_copy(zero_hbm.at[s], acc_sh.at[pl.ds(s * RPW, RPW)])
    plsc.subcore_barrier()

    @pl.loop(0, NCH, step=8)
    def _(j):
        copies = [pltpu.async_copy(ones_v, acc_sh.at[idx_v.at[j + k]], dsem,
                                   add=True) for k in range(8)]
        for cp in copies:
            cp.wait()

    plsc.subcore_barrier()
    pltpu.sync_copy(acc_sh.at[pl.ds(s * RPW, RPW)], out_hbm.at[c, s])


@functools.partial(
    pl.kernel,
    mesh=_MESH,
    out_type=jax.ShapeDtypeStruct((NC, NS, RPW, 32), jnp.float32),
    scratch_types=[
        pltpu.VMEM((NCH, CH), jnp.int32),
        pltpu.VMEM((NCH, CH), jnp.int32),
        pltpu.VMEM((8 * CH, 32), jnp.float32),
        pltpu.VMEM((8 * CH, 32), jnp.float32),
        pltpu.VMEM_SHARED((N, 32), jnp.float32),
        pltpu.SemaphoreType.DMA,
        pltpu.SemaphoreType.DMA,
        pltpu.SemaphoreType.DMA,
        pltpu.SemaphoreType.DMA,
    ],
    compiler_params=pltpu.CompilerParams(use_tc_tiling_on_sc=False, skip_device_barrier=True),
)
def _spmm_pass(hs_hbm, src_hbm, dst_hbm, zero_hbm, out_hbm,
               src_v, dst_v, buf_a, buf_b, acc_sh, gsem_a, gsem_b,
               ssem_a, ssem_b):
    c = lax.axis_index("c")
    s = lax.axis_index("s")
    pltpu.sync_copy(src_hbm.at[c, s], src_v)
    pltpu.sync_copy(dst_hbm.at[c, s], dst_v)
    rows = pl.ds(s * RPW, RPW)
    pltpu.sync_copy(zero_hbm.at[s], acc_sh.at[rows])
    plsc.subcore_barrier()

    def _fire_g(b, buf, sem):
        for k in range(8):
            pltpu.async_copy(hs_hbm.at[src_v.at[b * 8 + k]],
                             buf.at[pl.ds(k * CH, CH)], sem)

    def _fire_s(b, buf, sem):
        for k in range(8):
            pltpu.async_copy(buf.at[pl.ds(k * CH, CH)],
                             acc_sh.at[dst_v.at[b * 8 + k]], sem, add=True)

    def _drain(buf, sem):
        pltpu.make_async_copy(hs_hbm.at[pl.ds(0, 8 * CH)], buf, sem).wait()

    NB = NCH // 8
    _fire_g(0, buf_a, gsem_a)

    @pl.loop(0, NB, step=2)
    def _(b):
        _drain(buf_a, gsem_a)
        @pl.when(b + 1 < NB)
        def _():
            pl.when(b > 0)(lambda: _drain(buf_b, ssem_b))
            _fire_g(b + 1, buf_b, gsem_b)
        _fire_s(b, buf_a, ssem_a)
        @pl.when(b + 1 < NB)
        def _():
            _drain(buf_b, gsem_b)
            _drain(buf_a, ssem_a)
            pl.when(b + 2 < NB)(lambda: _fire_g(b + 2, buf_a, gsem_a))
            _fire_s(b + 1, buf_b, ssem_b)
    _drain(buf_b, ssem_b)

    plsc.subcore_barrier()
    pltpu.sync_copy(acc_sh.at[rows], out_hbm.at[c, s])


def _dinv_scale_body(deg_ref, x_ref, w_ref, dinv_ref, hs_ref):
    d = deg_ref[...]
    deg = d[0, :, 0:1] + d[1, :, 0:1] + 1.0
    dinv = lax.rsqrt(deg)
    dinv_ref[...] = dinv
    h = jnp.dot(x_ref[...], w_ref[...], preferred_element_type=jnp.float32)
    hs_ref[...] = h * dinv


def _layer_body(acc_ref, hs_ref, dinv_ref, b_ref, g_ref, be_ref, w_ref,
                o_ref):
    a = acc_ref[...]
    dinv = dinv_ref[...]
    t = (a[0] + a[1] + hs_ref[...]) * dinv + b_ref[...]
    t = jnp.maximum(t, 0.0)
    mu = jnp.mean(t, axis=1, keepdims=True)
    var = jnp.mean((t - mu) ** 2, axis=1, keepdims=True)
    t = (t - mu) * lax.rsqrt(var + 1e-5) * g_ref[...] + be_ref[...]
    o_ref[...] = jnp.dot(t, w_ref[...],
                         preferred_element_type=jnp.float32) * dinv


def _head_body(acc_ref, hs_ref, dinv_ref, b3_ref, wp1_ref, bp1_ref,
               wp2_ref, bp2_ref, emb_ref, o_ref):
    a = acc_ref[...]
    emb = (a[0] + a[1] + hs_ref[...]) * dinv_ref[...] + b3_ref[...]
    emb_ref[...] = emb
    h = jnp.maximum(emb, 0.0)
    h = jnp.dot(h, wp1_ref[...], preferred_element_type=jnp.float32)
    h = h + bp1_ref[...]
    q = jnp.dot(h, wp2_ref[...], preferred_element_type=jnp.float32)
    q = q + bp2_ref[...]
    m = jnp.max(q, axis=1, keepdims=True)
    shifted = q - m
    lse = jnp.log(jnp.sum(jnp.exp(shifted), axis=1, keepdims=True))
    o_ref[...] = shifted - lse


def _f32(*shape):
    return jax.ShapeDtypeStruct(shape, jnp.float32)


def kernel(x, edge_index, W1, b1, g1, be1, W2, b2, g2, be2, W3, b3,
           Wp1, bp1, Wp2, bp2):
    src_r = edge_index[0].reshape(NC, NS, NCH, CH)
    dst_r = edge_index[1].reshape(NC, NS, NCH, CH)
    ones16 = jnp.ones((CH, 16), jnp.float32)
    z16 = jnp.zeros((NS, RPW, 16), jnp.float32)
    z32 = jnp.zeros((NS, RPW, 32), jnp.float32)

    deg_acc = _deg_pass(dst_r, ones16, z16).reshape(NC, N, 16)
    dinv, hs1 = pl.pallas_call(
        _dinv_scale_body, out_shape=(_f32(N, 1), _f32(N, 32)))(deg_acc, x, W1)

    acc1 = _spmm_pass(hs1, src_r, dst_r, z32)
    hs2 = pl.pallas_call(_layer_body, out_shape=_f32(N, 32))(
        acc1.reshape(NC, N, 32), hs1, dinv, b1.reshape(1, 32),
        g1.reshape(1, 32), be1.reshape(1, 32), W2)

    acc2 = _spmm_pass(hs2, src_r, dst_r, z32)
    hs3 = pl.pallas_call(_layer_body, out_shape=_f32(N, 32))(
        acc2.reshape(NC, N, 32), hs2, dinv, b2.reshape(1, 32),
        g2.reshape(1, 32), be2.reshape(1, 32), W3)

    acc3 = _spmm_pass(hs3, src_r, dst_r, z32)
    emb, logp = pl.pallas_call(
        _head_body, out_shape=(_f32(N, 32), _f32(N, 128)))(
        acc3.reshape(NC, N, 32), hs3, dinv, b3.reshape(1, 32), Wp1,
        bp1.reshape(1, 32), Wp2, bp2.reshape(1, 128))
    return (emb, logp)


# speedup vs baseline: 1.0337x; 1.0133x over previous
import functools

import jax
import jax.numpy as jnp
from jax import lax
from jax.experimental import pallas as pl
from jax.experimental.pallas import tpu as pltpu
from jax.experimental.pallas import tpu_sc as plsc

N = 10000
E = 320000
NC = 2
NS = 16
EPW = E // (NC * NS)
CH = 125
NCH = EPW // CH
RPW = N // NS
KB = 4
NB = NCH // KB

_MESH = plsc.VectorSubcoreMesh(core_axis_name="c", subcore_axis_name="s")


@functools.partial(
    pl.kernel,
    mesh=_MESH,
    out_type=jax.ShapeDtypeStruct((NC, NS, RPW, 16), jnp.float32),
    scratch_types=[
        pltpu.VMEM((NCH, CH), jnp.int32),
        pltpu.VMEM((CH, 16), jnp.float32),
        pltpu.VMEM_SHARED((N, 16), jnp.float32),
        pltpu.SemaphoreType.DMA,
    ],
    compiler_params=pltpu.CompilerParams(use_tc_tiling_on_sc=False),
)
def _deg_pass(dst_hbm, ones_hbm, zero_hbm, out_hbm, idx_v, ones_v, acc_sh,
              dsem):
    c = lax.axis_index("c")
    s = lax.axis_index("s")
    pltpu.sync_copy(dst_hbm.at[c, s], idx_v)
    pltpu.sync_copy(ones_hbm, ones_v)
    pltpu.sync_copy(zero_hbm.at[s], acc_sh.at[pl.ds(s * RPW, RPW)])
    plsc.subcore_barrier()

    @pl.loop(0, NCH, step=8)
    def _(j):
        copies = [pltpu.async_copy(ones_v, acc_sh.at[idx_v.at[j + k]], dsem,
                                   add=True) for k in range(8)]
        for cp in copies:
            cp.wait()

    plsc.subcore_barrier()
    pltpu.sync_copy(acc_sh.at[pl.ds(s * RPW, RPW)], out_hbm.at[c, s])


@functools.partial(
    pl.kernel,
    mesh=_MESH,
    out_type=jax.ShapeDtypeStruct((NC, NS, RPW, 32), jnp.float32),
    scratch_types=[
        pltpu.VMEM((NCH, CH), jnp.int32),
        pltpu.VMEM((NCH, CH), jnp.int32),
        pltpu.VMEM((KB * CH, 32), jnp.float32),
        pltpu.VMEM((KB * CH, 32), jnp.float32),
        pltpu.VMEM((KB * CH, 32), jnp.float32),
        pltpu.VMEM((KB * CH, 32), jnp.float32),
        pltpu.VMEM_SHARED((N, 32), jnp.float32),
        pltpu.SemaphoreType.DMA,
        pltpu.SemaphoreType.DMA,
        pltpu.SemaphoreType.DMA,
        pltpu.SemaphoreType.DMA,
        pltpu.SemaphoreType.DMA,
        pltpu.SemaphoreType.DMA,
        pltpu.SemaphoreType.DMA,
        pltpu.SemaphoreType.DMA,
    ],
    compiler_params=pltpu.CompilerParams(use_tc_tiling_on_sc=False),
)
def _spmm_pass(hs_hbm, src_hbm, dst_hbm, zero_hbm, out_hbm,
               src_v, dst_v, b0, b1, b2, b3, acc_sh,
               g0, g1, g2, g3, s0, s1, s2, s3):
    c = lax.axis_index("c")
    s = lax.axis_index("s")
    pltpu.sync_copy(src_hbm.at[c, s], src_v)
    pltpu.sync_copy(dst_hbm.at[c, s], dst_v)
    rows = pl.ds(s * RPW, RPW)
    pltpu.sync_copy(zero_hbm.at[s], acc_sh.at[rows])
    plsc.subcore_barrier()

    BUFS = (b0, b1, b2, b3)
    GS = (g0, g1, g2, g3)
    SS = (s0, s1, s2, s3)

    def _fire_g(b, buf, sem):
        for k in range(KB):
            pltpu.async_copy(hs_hbm.at[src_v.at[b * KB + k]],
                             buf.at[pl.ds(k * CH, CH)], sem)

    def _fire_s(b, buf, sem):
        for k in range(KB):
            pltpu.async_copy(buf.at[pl.ds(k * CH, CH)],
                             acc_sh.at[dst_v.at[b * KB + k]], sem, add=True)

    def _drain(buf, sem):
        pltpu.make_async_copy(hs_hbm.at[pl.ds(0, KB * CH)], buf, sem).wait()

    for u in range(4):
        _fire_g(u, BUFS[u], GS[u])

    @pl.loop(0, NB, step=4)
    def _(b):
        for u in range(4):
            _drain(BUFS[u], GS[u])
            _fire_s(b + u, BUFS[u], SS[u])
        for u in range(4):
            def _refill(u=u):
                _drain(BUFS[u], SS[u])
                _fire_g(b + u + 4, BUFS[u], GS[u])
            pl.when(b + u + 4 < NB)(_refill)

    for u in range(4):
        _drain(BUFS[u], SS[u])

    plsc.subcore_barrier()
    pltpu.sync_copy(acc_sh.at[rows], out_hbm.at[c, s])


def _dinv_scale_body(deg_ref, x_ref, w_ref, dinv_ref, hs_ref):
    d = deg_ref[...]
    deg = d[0, :, 0:1] + d[1, :, 0:1] + 1.0
    dinv = lax.rsqrt(deg)
    dinv_ref[...] = dinv
    h = jnp.dot(x_ref[...], w_ref[...], preferred_element_type=jnp.float32)
    hs_ref[...] = h * dinv


def _layer_body(acc_ref, hs_ref, dinv_ref, b_ref, g_ref, be_ref, w_ref,
                o_ref):
    a = acc_ref[...]
    dinv = dinv_ref[...]
    t = (a[0] + a[1] + hs_ref[...]) * dinv + b_ref[...]
    t = jnp.maximum(t, 0.0)
    mu = jnp.mean(t, axis=1, keepdims=True)
    var = jnp.mean((t - mu) ** 2, axis=1, keepdims=True)
    t = (t - mu) * lax.rsqrt(var + 1e-5) * g_ref[...] + be_ref[...]
    o_ref[...] = jnp.dot(t, w_ref[...],
                         preferred_element_type=jnp.float32) * dinv


def _head_body(acc_ref, hs_ref, dinv_ref, b3_ref, wp1_ref, bp1_ref,
               wp2_ref, bp2_ref, emb_ref, o_ref):
    a = acc_ref[...]
    emb = (a[0] + a[1] + hs_ref[...]) * dinv_ref[...] + b3_ref[...]
    emb_ref[...] = emb
    h = jnp.maximum(emb, 0.0)
    h = jnp.dot(h, wp1_ref[...], preferred_element_type=jnp.float32)
    h = h + bp1_ref[...]
    q = jnp.dot(h, wp2_ref[...], preferred_element_type=jnp.float32)
    q = q + bp2_ref[...]
    m = jnp.max(q, axis=1, keepdims=True)
    shifted = q - m
    lse = jnp.log(jnp.sum(jnp.exp(shifted), axis=1, keepdims=True))
    o_ref[...] = shifted - lse


def _f32(*shape):
    return jax.ShapeDtypeStruct(shape, jnp.float32)


def kernel(x, edge_index, W1, b1, g1, be1, W2, b2, g2, be2, W3, b3,
           Wp1, bp1, Wp2, bp2):
    src_r = edge_index[0].reshape(NC, NS, NCH, CH)
    dst_r = edge_index[1].reshape(NC, NS, NCH, CH)
    ones16 = jnp.ones((CH, 16), jnp.float32)
    z16 = jnp.zeros((NS, RPW, 16), jnp.float32)
    z32 = jnp.zeros((NS, RPW, 32), jnp.float32)

    deg_acc = _deg_pass(dst_r, ones16, z16).reshape(NC, N, 16)
    dinv, hs1 = pl.pallas_call(
        _dinv_scale_body, out_shape=(_f32(N, 1), _f32(N, 32)))(deg_acc, x, W1)

    acc1 = _spmm_pass(hs1, src_r, dst_r, z32)
    hs2 = pl.pallas_call(_layer_body, out_shape=_f32(N, 32))(
        acc1.reshape(NC, N, 32), hs1, dinv, b1.reshape(1, 32),
        g1.reshape(1, 32), be1.reshape(1, 32), W2)

    acc2 = _spmm_pass(hs2, src_r, dst_r, z32)
    hs3 = pl.pallas_call(_layer_body, out_shape=_f32(N, 32))(
        acc2.reshape(NC, N, 32), hs2, dinv, b2.reshape(1, 32),
        g2.reshape(1, 32), be2.reshape(1, 32), W3)

    acc3 = _spmm_pass(hs3, src_r, dst_r, z32)
    emb, logp = pl.pallas_call(
        _head_body, out_shape=(_f32(N, 32), _f32(N, 128)))(
        acc3.reshape(NC, N, 32), hs3, dinv, b3.reshape(1, 32), Wp1,
        bp1.reshape(1, 32), Wp2, bp2.reshape(1, 128))
    return (emb, logp)


# speedup vs baseline: 1.0839x; 1.0486x over previous
import functools

import jax
import jax.numpy as jnp
from jax import lax
from jax.experimental import pallas as pl
from jax.experimental.pallas import tpu as pltpu
from jax.experimental.pallas import tpu_sc as plsc

N = 10000
E = 320000
NC = 2
NS = 16
EPW = E // (NC * NS)
CH = 125
NCH = EPW // CH
RPW = N // NS
KB = 4
NB = NCH // KB

_MESH = plsc.VectorSubcoreMesh(core_axis_name="c", subcore_axis_name="s")


@functools.partial(
    pl.kernel,
    mesh=_MESH,
    out_type=jax.ShapeDtypeStruct((NC, NS, RPW, 16), jnp.float32),
    scratch_types=[
        pltpu.VMEM((NCH, CH), jnp.int32),
        pltpu.VMEM((CH, 16), jnp.float32),
        pltpu.VMEM_SHARED((N, 16), jnp.float32),
        pltpu.SemaphoreType.DMA,
    ],
    compiler_params=pltpu.CompilerParams(use_tc_tiling_on_sc=False),
)
def _deg_pass(dst_hbm, ones_hbm, zero_hbm, out_hbm, idx_v, ones_v, acc_sh,
              dsem):
    c = lax.axis_index("c")
    s = lax.axis_index("s")
    pltpu.sync_copy(dst_hbm.at[c, s], idx_v)
    pltpu.sync_copy(ones_hbm, ones_v)
    pltpu.sync_copy(zero_hbm.at[s], acc_sh.at[pl.ds(s * RPW, RPW)])
    plsc.subcore_barrier()

    @pl.loop(0, NCH, step=8)
    def _(j):
        copies = [pltpu.async_copy(ones_v, acc_sh.at[idx_v.at[j + k]], dsem,
                                   add=True) for k in range(8)]
        for cp in copies:
            cp.wait()

    plsc.subcore_barrier()
    pltpu.sync_copy(acc_sh.at[pl.ds(s * RPW, RPW)], out_hbm.at[c, s])


@functools.partial(
    pl.kernel,
    mesh=_MESH,
    out_type=jax.ShapeDtypeStruct((NC, NS, RPW, 32), jnp.float32),
    scratch_types=[
        pltpu.VMEM((NCH, CH), jnp.int32),
        pltpu.VMEM((NCH, CH), jnp.int32),
        pltpu.VMEM((KB * CH, 32), jnp.float32),
        pltpu.VMEM((KB * CH, 32), jnp.float32),
        pltpu.VMEM((KB * CH, 32), jnp.float32),
        pltpu.VMEM((KB * CH, 32), jnp.float32),
        pltpu.VMEM_SHARED((N, 32), jnp.float32),
        pltpu.SemaphoreType.DMA,
        pltpu.SemaphoreType.DMA,
        pltpu.SemaphoreType.DMA,
        pltpu.SemaphoreType.DMA,
        pltpu.SemaphoreType.DMA,
        pltpu.SemaphoreType.DMA,
        pltpu.SemaphoreType.DMA,
        pltpu.SemaphoreType.DMA,
    ],
    compiler_params=pltpu.CompilerParams(use_tc_tiling_on_sc=False),
)
def _spmm_pass(hs_hbm, src_hbm, dst_hbm, zero_hbm, out_hbm,
               src_v, dst_v, b0, b1, b2, b3, acc_sh,
               g0, g1, g2, g3, s0, s1, s2, s3):
    c = lax.axis_index("c")
    s = lax.axis_index("s")
    pltpu.sync_copy(src_hbm.at[c, s], src_v)
    pltpu.sync_copy(dst_hbm.at[c, s], dst_v)
    rows = pl.ds(s * RPW, RPW)
    pltpu.sync_copy(zero_hbm.at[s], acc_sh.at[rows])
    plsc.subcore_barrier()

    BUFS = (b0, b1, b2, b3)
    GS = (g0, g1, g2, g3)
    SS = (s0, s1, s2, s3)

    def _fire_g(b, buf, sem):
        for k in range(KB):
            pltpu.async_copy(hs_hbm.at[src_v.at[b * KB + k]],
                             buf.at[pl.ds(k * CH, CH)], sem)

    def _fire_s(b, buf, sem):
        for k in range(KB):
            pltpu.async_copy(buf.at[pl.ds(k * CH, CH)],
                             acc_sh.at[dst_v.at[b * KB + k]], sem, add=True)

    def _drain(buf, sem):
        pltpu.make_async_copy(hs_hbm.at[pl.ds(0, KB * CH)], buf, sem).wait()

    for u in range(4):
        _fire_g(u, BUFS[u], GS[u])

    @pl.loop(0, NB, step=4)
    def _(b):
        for u in range(4):
            _drain(BUFS[u], GS[u])
            _fire_s(b + u, BUFS[u], SS[u])
            _drain(BUFS[u], SS[u])

            def _refill(u=u):
                _fire_g(b + u + 4, BUFS[u], GS[u])
            pl.when(b + u + 4 < NB)(_refill)

    plsc.subcore_barrier()
    pltpu.sync_copy(acc_sh.at[rows], out_hbm.at[c, s])


def _dinv_scale_body(deg_ref, x_ref, w_ref, dinv_ref, hs_ref):
    d = deg_ref[...]
    deg = d[0, :, 0:1] + d[1, :, 0:1] + 1.0
    dinv = lax.rsqrt(deg)
    dinv_ref[...] = dinv
    h = jnp.dot(x_ref[...], w_ref[...], preferred_element_type=jnp.float32)
    hs_ref[...] = h * dinv


def _layer_body(acc_ref, hs_ref, dinv_ref, b_ref, g_ref, be_ref, w_ref,
                o_ref):
    a = acc_ref[...]
    dinv = dinv_ref[...]
    t = (a[0] + a[1] + hs_ref[...]) * dinv + b_ref[...]
    t = jnp.maximum(t, 0.0)
    mu = jnp.mean(t, axis=1, keepdims=True)
    var = jnp.mean((t - mu) ** 2, axis=1, keepdims=True)
    t = (t - mu) * lax.rsqrt(var + 1e-5) * g_ref[...] + be_ref[...]
    o_ref[...] = jnp.dot(t, w_ref[...],
                         preferred_element_type=jnp.float32) * dinv


def _head_body(acc_ref, hs_ref, dinv_ref, b3_ref, wp1_ref, bp1_ref,
               wp2_ref, bp2_ref, emb_ref, o_ref):
    a = acc_ref[...]
    emb = (a[0] + a[1] + hs_ref[...]) * dinv_ref[...] + b3_ref[...]
    emb_ref[...] = emb
    h = jnp.maximum(emb, 0.0)
    h = jnp.dot(h, wp1_ref[...], preferred_element_type=jnp.float32)
    h = h + bp1_ref[...]
    q = jnp.dot(h, wp2_ref[...], preferred_element_type=jnp.float32)
    q = q + bp2_ref[...]
    m = jnp.max(q, axis=1, keepdims=True)
    shifted = q - m
    lse = jnp.log(jnp.sum(jnp.exp(shifted), axis=1, keepdims=True))
    o_ref[...] = shifted - lse


def _f32(*shape):
    return jax.ShapeDtypeStruct(shape, jnp.float32)


def kernel(x, edge_index, W1, b1, g1, be1, W2, b2, g2, be2, W3, b3,
           Wp1, bp1, Wp2, bp2):
    src_r = edge_index[0].reshape(NC, NS, NCH, CH)
    dst_r = edge_index[1].reshape(NC, NS, NCH, CH)
    ones16 = jnp.ones((CH, 16), jnp.float32)
    z16 = jnp.zeros((NS, RPW, 16), jnp.float32)
    z32 = jnp.zeros((NS, RPW, 32), jnp.float32)

    deg_acc = _deg_pass(dst_r, ones16, z16).reshape(NC, N, 16)
    dinv, hs1 = pl.pallas_call(
        _dinv_scale_body, out_shape=(_f32(N, 1), _f32(N, 32)))(deg_acc, x, W1)

    acc1 = _spmm_pass(hs1, src_r, dst_r, z32)
    hs2 = pl.pallas_call(_layer_body, out_shape=_f32(N, 32))(
        acc1.reshape(NC, N, 32), hs1, dinv, b1.reshape(1, 32),
        g1.reshape(1, 32), be1.reshape(1, 32), W2)

    acc2 = _spmm_pass(hs2, src_r, dst_r, z32)
    hs3 = pl.pallas_call(_layer_body, out_shape=_f32(N, 32))(
        acc2.reshape(NC, N, 32), hs2, dinv, b2.reshape(1, 32),
        g2.reshape(1, 32), be2.reshape(1, 32), W3)

    acc3 = _spmm_pass(hs3, src_r, dst_r, z32)
    emb, logp = pl.pallas_call(
        _head_body, out_shape=(_f32(N, 32), _f32(N, 128)))(
        acc3.reshape(NC, N, 32), hs3, dinv, b3.reshape(1, 32), Wp1,
        bp1.reshape(1, 32), Wp2, bp2.reshape(1, 128))
    return (emb, logp)
